# baseline (speedup 1.0000x reference)
import functools

import jax
import jax.numpy as jnp
from jax import lax
from jax.experimental import pallas as pl
from jax.experimental.pallas import tpu as pltpu
from jax.experimental.pallas import tpu_sc as plsc

NC = 2
NS = 16
LANES = 16
NW = NC * NS
CHUNK = 128
NG = 4
NI = 8


def _embed_kernel(n_fields, vocab, dim, batch):
    pack = 128 // dim
    pack_shift = pack.bit_length() - 1
    win_per_w = (batch // CHUNK) // NW
    n_blocks = (n_fields + pack - 1) // pack
    rem_f = n_fields - (n_blocks - 1) * pack
    mesh = plsc.VectorSubcoreMesh(core_axis_name="c", subcore_axis_name="s")

    @functools.partial(
        pl.kernel,
        mesh=mesh,
        compiler_params=pltpu.CompilerParams(
            use_tc_tiling_on_sc=True, needs_layout_passes=False),
        out_type=jax.ShapeDtypeStruct((batch, n_blocks * 128), jnp.float32),
        scratch_types=(
            [pltpu.VMEM((CHUNK,), jnp.int32) for _ in range(NI + NG)]
            + [pltpu.VMEM((CHUNK, 128), jnp.float32) for _ in range(NG)]
            + [pltpu.VMEM((CHUNK, 128), jnp.float32),
               pltpu.VMEM((CHUNK, 128), jnp.float32)]
            + [pltpu.SemaphoreType.DMA for _ in range(NI + NG + 2)]
        ),
    )
    def k(tables_hbm, featsT_hbm, out_hbm, *refs):
        idx_v = refs[:NI]
        idxp_v = refs[NI:NI + NG]
        rows = refs[NI + NG:NI + 2 * NG]
        blks = refs[NI + 2 * NG:NI + 2 * NG + 2]
        sems = refs[NI + 2 * NG + 2:]
        isem = sems[:NI]
        gsem = sems[NI:NI + NG]
        osem = sems[NI + NG:]

        wid = lax.axis_index("s") * NC + lax.axis_index("c")

        def b0_of(w_):
            return pl.multiple_of((wid * win_per_w + w_) * CHUNK, CHUNK)

        def stage_idx(f, w_, s):
            pltpu.make_async_copy(
                featsT_hbm.at[f, pl.ds(b0_of(w_), CHUNK)], idx_v[s], isem[s],
            ).start()

        def fire_gather(f, s, g):
            pltpu.make_async_copy(
                featsT_hbm.at[0, pl.ds(0, CHUNK)], idx_v[s], isem[s],
            ).wait()
            for t in range(CHUNK // LANES):
                sl = pl.ds(t * LANES, LANES)
                idxp_v[g][sl] = jnp.right_shift(idx_v[s][sl], pack_shift)
            pltpu.make_async_copy(
                tables_hbm.at[f].at[idxp_v[g]], rows[g], gsem[g],
            ).start()

        def blk_of(t):
            return blks[t % 2]

        def out_desc(t, w_):
            return pltpu.make_async_copy(
                blk_of(t),
                out_hbm.at[pl.ds(b0_of(w_), CHUNK),
                           pl.ds(pl.multiple_of(t * 128, 128), 128)],
                osem[t % 2],
            )

        def extract(s, g, t, q):
            blk_ref = blk_of(t)

            lanev = jnp.arange(LANES, dtype=jnp.int32)

            def body(g16, carry):
                base = g16 * LANES
                off16 = (idx_v[s][pl.ds(base, LANES)] & (pack - 1)) * dim
                r16 = lanev + base
                for j in range(dim):
                    vals = plsc.load_gather(rows[g], [r16, off16 + j])
                    plsc.store_scatter(
                        blk_ref,
                        [r16, jnp.full((LANES,), q * dim + j, jnp.int32)],
                        vals)
                return carry

            lax.fori_loop(0, CHUNK // LANES, body, 0)

        def window(w_, carry):
            for s in range(NI):
                stage_idx(s, w_, s)
            for c in range(NG):
                fire_gather(c, c % NI, c)
            for c in range(n_fields):
                t, q = c // pack, c % pack
                sl, g = c % NI, c % NG
                pltpu.make_async_copy(
                    tables_hbm.at[0].at[idxp_v[g]], rows[g], gsem[g],
                ).wait()

                if q == 0:
                    if t >= 2:
                        out_desc(t - 2, w_).wait()
                    else:
                        tp = n_blocks - 1 if t == (n_blocks - 1) % 2 \
                            else n_blocks - 2
                        @pl.when(w_ > 0)
                        def _():
                            out_desc(tp, w_ - 1).wait()

                extract(sl, g, t, q)

                if q == pack - 1 or c == n_fields - 1:
                    out_desc(t, w_).start()

                ci = c + NI
                if ci < n_fields:
                    stage_idx(ci, w_, ci % NI)
                cg = c + NG
                if cg < n_fields:
                    fire_gather(cg, cg % NI, cg % NG)
            return carry

        lax.fori_loop(0, win_per_w, window, 0)
        for t in (n_blocks - 2, n_blocks - 1):
            out_desc(t, win_per_w - 1).wait()

    return k


def kernel(features, tables):
    b, f = features.shape
    f2, vocab, dim = tables.shape
    assert f == f2
    pack = 128 // dim
    assert 128 % dim == 0 and vocab % pack == 0 and b % CHUNK == 0
    assert (b // CHUNK) % NW == 0 and dim % LANES == 0 and f > 2 * pack

    feats_t = features.astype(jnp.int32).T
    tables_p = tables.reshape(f, vocab // pack, 128)
    out = _embed_kernel(f, vocab, dim, b)(tables_p, feats_t)
    return out[:, : f * dim]


# speedup vs baseline: 1.3184x; 1.3184x over previous
import functools

import jax
import jax.numpy as jnp
from jax import lax
from jax.experimental import pallas as pl
from jax.experimental.pallas import tpu as pltpu
from jax.experimental.pallas import tpu_sc as plsc

NC = 2
NS = 16
NW = NC * NS
CHUNK = 128
NBUF = 8


def _embed_kernel(n_fields, vocab, dim, batch):
    n_rows = n_fields * batch
    chunks_per_field = batch // CHUNK
    per_w = (n_rows // CHUNK) // NW
    n_groups = per_w // NBUF
    mesh = plsc.VectorSubcoreMesh(core_axis_name="c", subcore_axis_name="s")

    @functools.partial(
        pl.kernel,
        mesh=mesh,
        compiler_params=pltpu.CompilerParams(use_tc_tiling_on_sc=False),
        out_type=jax.ShapeDtypeStruct((batch, n_fields * dim), jnp.float32),
        scratch_types=(
            [pltpu.VMEM((NBUF, CHUNK), jnp.int32)]
            + [pltpu.VMEM((CHUNK, dim), jnp.float32) for _ in range(NBUF)]
            + [pltpu.SemaphoreType.DMA for _ in range(3 * NBUF)]
        ),
    )
    def k(tables_hbm, featsT_hbm, out_hbm, idx_v, *bufs_sems):
        rows = bufs_sems[:NBUF]
        gsem = bufs_sems[NBUF:2 * NBUF]
        osem = bufs_sems[2 * NBUF:3 * NBUF]
        isem = bufs_sems[3 * NBUF:]

        wid = lax.axis_index("s") * NC + lax.axis_index("c")
        c0 = wid * per_w

        def stage_idx(k_, b):
            c = c0 + k_
            f = c // chunks_per_field
            b0 = (c % chunks_per_field) * CHUNK
            pltpu.make_async_copy(
                featsT_hbm.at[f, pl.ds(b0, CHUNK)], idx_v.at[b], isem[b],
            ).start()

        def idx_wait(b):
            pltpu.make_async_copy(
                featsT_hbm.at[0, pl.ds(0, CHUNK)], idx_v.at[b], isem[b],
            ).wait()

        def out_slice(k_):
            c = c0 + k_
            f = c // chunks_per_field
            b0 = (c % chunks_per_field) * CHUNK
            return out_hbm.at[pl.ds(b0, CHUNK), pl.ds(f * dim, dim)]

        def gather(k_, b):
            c = c0 + k_
            f = c // chunks_per_field
            pltpu.make_async_copy(
                tables_hbm.at[f].at[idx_v.at[b]], rows[b], gsem[b],
            ).start()

        def drain_and_put(k_, b):
            pltpu.make_async_copy(
                tables_hbm.at[0].at[idx_v.at[b]], rows[b], gsem[b],
            ).wait()
            pltpu.make_async_copy(rows[b], out_slice(k_), osem[b]).start()
            @pl.when(k_ + NBUF < per_w)
            def _():
                stage_idx(k_ + NBUF, b)

        def out_wait(k_, b):
            pltpu.make_async_copy(rows[b], out_slice(k_), osem[b]).wait()

        for b in range(NBUF):
            stage_idx(b, b)

        def group(g, carry):
            for b in range(NBUF):
                k_ = g * NBUF + b

                @pl.when(g > 0)
                def _():
                    out_wait(k_ - NBUF, b)

                idx_wait(b)
                gather(k_, b)
            for b in range(NBUF):
                drain_and_put(g * NBUF + b, b)
            return carry

        lax.fori_loop(0, n_groups, group, 0)
        for b in range(NBUF):
            out_wait((n_groups - 1) * NBUF + b, b)

    return k


def kernel(features, tables):
    b, f = features.shape
    f2, vocab, dim = tables.shape
    assert f == f2
    n_chunks = b * f // CHUNK
    assert b % CHUNK == 0 and n_chunks % (NW * NBUF) == 0

    feats_t = features.astype(jnp.int32).T
    return _embed_kernel(f, vocab, dim, b)(tables, feats_t)
